# baseline (speedup 1.0000x reference)
import functools

import jax
import jax.numpy as jnp
from jax import lax
from jax.experimental import pallas as pl
from jax.experimental.pallas import tpu as pltpu
from jax.experimental.pallas import tpu_sc as plsc

N = 10000
E = 320000
D = 128
DE = 16
DG = 16
LATENT = 128

NC = 2
NS = 16
NW = NC * NS

EPW = E // NW
CH = 80
NCHUNK = EPW // CH
NDT = 10
NPT = N // NDT

BN = 200
BE = 1600

_SC_MESH = plsc.VectorSubcoreMesh(core_axis_name="c", subcore_axis_name="s")


def _proj_body(x_ref, wa_ref, wb_ref, px_ref, qx_ref):
    x = x_ref[...]
    px_ref[...] = jnp.dot(x, wa_ref[...], preferred_element_type=jnp.float32)
    qx_ref[...] = jnp.dot(x, wb_ref[...], preferred_element_type=jnp.float32)


def _project(x, wa, wb):
    grid = (N // BN,)
    return pl.pallas_call(
        _proj_body,
        grid=grid,
        in_specs=[
            pl.BlockSpec((BN, D), lambda i: (i, 0)),
            pl.BlockSpec((D, LATENT), lambda i: (0, 0)),
            pl.BlockSpec((D, LATENT), lambda i: (0, 0)),
        ],
        out_specs=[
            pl.BlockSpec((BN, LATENT), lambda i: (i, 0)),
            pl.BlockSpec((BN, LATENT), lambda i: (i, 0)),
        ],
        out_shape=[
            jax.ShapeDtypeStruct((N, LATENT), jnp.float32),
            jax.ShapeDtypeStruct((N, LATENT), jnp.float32),
        ],
    )(x, wa, wb)


NBUF = 5
GRPE = NW * NBUF * CH


@functools.lru_cache(maxsize=None)
def _make_gather(ne):
    epw = ne // NW
    ngrp = epw // (NBUF * CH)

    @functools.partial(
        pl.kernel,
        out_type=jax.ShapeDtypeStruct((ne, LATENT), jnp.float32),
        mesh=_SC_MESH,
        scratch_types=[
            pltpu.VMEM((NBUF, CH), jnp.int32),
            pltpu.VMEM((NBUF, CH), jnp.int32),
            pltpu.VMEM((NBUF, CH, LATENT), jnp.float32),
            pltpu.SemaphoreType.DMA((NBUF,)),
            pltpu.SemaphoreType.DMA((NBUF,)),
            pltpu.SemaphoreType.DMA((NBUF,)),
        ],
    )
    def _gather_sum(px_hbm, qx_hbm, row_hbm, col_hbm, s_hbm,
                    idxa, idxb, bufs, sema, semb, semw):
        c = lax.axis_index("c")
        s = lax.axis_index("s")
        wid = s * NC + c
        base0 = wid * epw

        def group(g, carry):
            gbase = base0 + g * NBUF * CH
            for b in range(NBUF):
                base = gbase + b * CH

                @pl.when(g > 0)
                def _():
                    pltpu.make_async_copy(
                        bufs.at[b], s_hbm.at[pl.ds(base - NBUF * CH, CH)],
                        semw.at[b]).wait()

                pltpu.sync_copy(row_hbm.at[pl.ds(base, CH)], idxa.at[b])
                pltpu.sync_copy(col_hbm.at[pl.ds(base, CH)], idxb.at[b])
                pltpu.async_copy(px_hbm.at[idxa.at[b]], bufs.at[b], sema.at[b])
            for b in range(NBUF):
                pltpu.make_async_copy(px_hbm.at[idxa.at[b]], bufs.at[b],
                                      sema.at[b]).wait()
                pltpu.async_copy(qx_hbm.at[idxb.at[b]], bufs.at[b], semb.at[b],
                                 add=True)
            for b in range(NBUF):
                base = gbase + b * CH
                pltpu.make_async_copy(qx_hbm.at[idxb.at[b]], bufs.at[b],
                                      semb.at[b]).wait()
                pltpu.async_copy(bufs.at[b], s_hbm.at[pl.ds(base, CH)],
                                 semw.at[b])
            return carry

        lax.fori_loop(0, ngrp, group, 0)
        for b in range(NBUF):
            base = base0 + (ngrp - 1) * NBUF * CH + b * CH
            pltpu.make_async_copy(bufs.at[b], s_hbm.at[pl.ds(base, CH)],
                                  semw.at[b]).wait()

    return _gather_sum


def _edge_body(s_ref, ea_ref, wc_ref, wd_ref, be1_ref, w2_ref, b2_ref, g_ref,
               eo_ref, esum_ref):
    pid = pl.program_id(0)
    c0 = jnp.dot(g_ref[...], wd_ref[...], preferred_element_type=jnp.float32) + be1_ref[...]
    h = (s_ref[...]
         + jnp.dot(ea_ref[...], wc_ref[...], preferred_element_type=jnp.float32) + c0)
    h = jnp.maximum(h, 0.0)
    eo = jnp.dot(h, w2_ref[...], preferred_element_type=jnp.float32) + b2_ref[...]
    eo_ref[...] = eo

    @pl.when(pid == 0)
    def _():
        esum_ref[...] = jnp.zeros_like(esum_ref)

    esum_ref[...] += jnp.sum(eo, axis=0, keepdims=True)


def _edge_mlp(s, edge_attr, wc, wd, be1, w2, b2, g):
    ne = s.shape[0]
    grid = (ne // BE,)
    return pl.pallas_call(
        _edge_body,
        grid=grid,
        in_specs=[
            pl.BlockSpec((BE, LATENT), lambda i: (i, 0)),
            pl.BlockSpec((BE, DE), lambda i: (i, 0)),
            pl.BlockSpec((DE, LATENT), lambda i: (0, 0)),
            pl.BlockSpec((DG, LATENT), lambda i: (0, 0)),
            pl.BlockSpec((1, LATENT), lambda i: (0, 0)),
            pl.BlockSpec((LATENT, DE), lambda i: (0, 0)),
            pl.BlockSpec((1, DE), lambda i: (0, 0)),
            pl.BlockSpec((1, DG), lambda i: (0, 0)),
        ],
        out_specs=[
            pl.BlockSpec((BE, DE), lambda i: (i, 0)),
            pl.BlockSpec((1, DE), lambda i: (0, 0)),
        ],
        out_shape=[
            jax.ShapeDtypeStruct((ne, DE), jnp.float32),
            jax.ShapeDtypeStruct((1, DE), jnp.float32),
        ],
    )(s, edge_attr, wc, wd, be1, w2, b2, g)


@functools.lru_cache(maxsize=None)
def _make_scatter(ne):
    epw = ne // NW
    ngrp = epw // (NBUF * CH)

    @functools.partial(
        pl.kernel,
        out_type=[
            jax.ShapeDtypeStruct((NC, N, DE), jnp.float32),
            jax.ShapeDtypeStruct((NC, N, DE), jnp.float32),
        ],
        mesh=_SC_MESH,
        compiler_params=pltpu.CompilerParams(use_tc_tiling_on_sc=False),
        scratch_types=[
            pltpu.VMEM((NBUF, CH), jnp.int32),
            pltpu.VMEM((NBUF, CH, DE), jnp.float32),
            pltpu.VMEM((CH, DE), jnp.float32),
            pltpu.VMEM((NPT, DE), jnp.float32),
            pltpu.VMEM_SHARED((N, DE), jnp.float32),
            pltpu.VMEM_SHARED((N, DE), jnp.float32),
            pltpu.SemaphoreType.DMA((NBUF,)),
            pltpu.SemaphoreType.DMA((NBUF,)),
            pltpu.SemaphoreType.DMA((NBUF,)),
            pltpu.SemaphoreType.DMA((NBUF,)),
        ],
    )
    def _segment_sum(eo_hbm, col_hbm, agg_hbm, cnt_hbm,
                     idxs, vals, onesbuf, stage, agg_sh, cnt_sh,
                     semi, semv, semadd, semone):
        c = lax.axis_index("c")
        s = lax.axis_index("s")
        wid = s * NC + c
        base0 = wid * epw

        zeros16 = jnp.zeros((16,), jnp.float32)
        ones16 = jnp.ones((16,), jnp.float32)

        def orow(i, carry):
            onesbuf[i, :] = ones16
            return carry

        lax.fori_loop(0, CH, orow, 0, unroll=4)

        @pl.when(s < NDT)
        def _():
            def zrow(i, carry):
                stage[i, :] = zeros16
                return carry

            lax.fori_loop(0, NPT, zrow, 0, unroll=4)
            pltpu.sync_copy(stage, agg_sh.at[pl.ds(s * NPT, NPT)])
            pltpu.sync_copy(stage, cnt_sh.at[pl.ds(s * NPT, NPT)])

        plsc.subcore_barrier()

        def group(g, carry):
            gbase = base0 + g * NBUF * CH
            for b in range(NBUF):
                base = gbase + b * CH

                @pl.when(g > 0)
                def _():
                    pltpu.make_async_copy(
                        vals.at[b], agg_sh.at[idxs.at[b]], semadd.at[b]).wait()
                    pltpu.make_async_copy(
                        onesbuf, cnt_sh.at[idxs.at[b]], semone.at[b]).wait()

                pltpu.async_copy(col_hbm.at[pl.ds(base, CH)], idxs.at[b],
                                 semi.at[b])
                pltpu.async_copy(eo_hbm.at[pl.ds(base, CH)], vals.at[b],
                                 semv.at[b])
            for b in range(NBUF):
                base = gbase + b * CH
                pltpu.make_async_copy(col_hbm.at[pl.ds(base, CH)], idxs.at[b],
                                      semi.at[b]).wait()
                pltpu.make_async_copy(eo_hbm.at[pl.ds(base, CH)], vals.at[b],
                                      semv.at[b]).wait()
                pltpu.async_copy(vals.at[b], agg_sh.at[idxs.at[b]],
                                 semadd.at[b], add=True)
                pltpu.async_copy(onesbuf, cnt_sh.at[idxs.at[b]],
                                 semone.at[b], add=True)
            return carry

        lax.fori_loop(0, ngrp, group, 0)
        for b in range(NBUF):
            pltpu.make_async_copy(vals.at[b], agg_sh.at[idxs.at[b]],
                                  semadd.at[b]).wait()
            pltpu.make_async_copy(onesbuf, cnt_sh.at[idxs.at[b]],
                                  semone.at[b]).wait()
        plsc.subcore_barrier()

        @pl.when(s < NDT)
        def _():
            pltpu.sync_copy(agg_sh.at[pl.ds(s * NPT, NPT)], stage)
            pltpu.sync_copy(stage, agg_hbm.at[c, pl.ds(s * NPT, NPT)])
            pltpu.sync_copy(cnt_sh.at[pl.ds(s * NPT, NPT)], stage)
            pltpu.sync_copy(stage, cnt_hbm.at[c, pl.ds(s * NPT, NPT)])

    return _segment_sum


def _node_body(x_ref, aggp_ref, cntp_ref, aggp2_ref, cntp2_ref,
               wn1x_ref, wn1a_ref, wn1g_ref,
               bn1_ref, wn2_ref, bn2_ref, esum_ref, esum2_ref, g_ref,
               wg1g_ref, wg1n_ref, wg1e_ref, bg1_ref, wg2_ref, bg2_ref,
               no_ref, nsum_ref, gout_ref):
    pid = pl.program_id(0)
    nblocks = pl.num_programs(0)
    ap = aggp_ref[0] + aggp_ref[1] + aggp2_ref[0] + aggp2_ref[1]
    cp = cntp_ref[0] + cntp_ref[1] + cntp2_ref[0] + cntp2_ref[1]
    agg = ap / jnp.maximum(cp, 1.0)
    cn0 = jnp.dot(g_ref[...], wn1g_ref[...], preferred_element_type=jnp.float32) + bn1_ref[...]
    h = (jnp.dot(x_ref[...], wn1x_ref[...], preferred_element_type=jnp.float32)
         + jnp.dot(agg, wn1a_ref[...], preferred_element_type=jnp.float32) + cn0)
    h = jnp.maximum(h, 0.0)
    no = jnp.dot(h, wn2_ref[...], preferred_element_type=jnp.float32) + bn2_ref[...]
    no_ref[...] = no

    @pl.when(pid == 0)
    def _():
        nsum_ref[...] = jnp.zeros_like(nsum_ref)

    nsum_ref[...] += jnp.sum(no, axis=0, keepdims=True)

    @pl.when(pid == nblocks - 1)
    def _():
        node_mean = nsum_ref[...] * (1.0 / N)
        edge_mean = (esum_ref[...] + esum2_ref[...]) * (1.0 / E)
        hg = (jnp.dot(g_ref[...], wg1g_ref[...], preferred_element_type=jnp.float32)
              + jnp.dot(node_mean, wg1n_ref[...], preferred_element_type=jnp.float32)
              + jnp.dot(edge_mean, wg1e_ref[...], preferred_element_type=jnp.float32)
              + bg1_ref[...])
        hg = jnp.maximum(hg, 0.0)
        gout_ref[...] = jnp.dot(hg, wg2_ref[...], preferred_element_type=jnp.float32) + bg2_ref[...]


def _node_global(x, aggp, cntp, aggp2, cntp2, wn1x, wn1a, wn1g, bn1, wn2, bn2,
                 esum, esum2, g, wg1g, wg1n, wg1e, bg1, wg2, bg2):
    grid = (N // BN,)
    const = lambda i: (0, 0)
    return pl.pallas_call(
        _node_body,
        grid=grid,
        in_specs=[
            pl.BlockSpec((BN, D), lambda i: (i, 0)),
            pl.BlockSpec((NC, BN, DE), lambda i: (0, i, 0)),
            pl.BlockSpec((NC, BN, DE), lambda i: (0, i, 0)),
            pl.BlockSpec((NC, BN, DE), lambda i: (0, i, 0)),
            pl.BlockSpec((NC, BN, DE), lambda i: (0, i, 0)),
            pl.BlockSpec((D, LATENT), const),
            pl.BlockSpec((DE, LATENT), const),
            pl.BlockSpec((DG, LATENT), const),
            pl.BlockSpec((1, LATENT), const),
            pl.BlockSpec((LATENT, D), const),
            pl.BlockSpec((1, D), const),
            pl.BlockSpec((1, DE), const),
            pl.BlockSpec((1, DE), const),
            pl.BlockSpec((1, DG), const),
            pl.BlockSpec((DG, LATENT), const),
            pl.BlockSpec((D, LATENT), const),
            pl.BlockSpec((DE, LATENT), const),
            pl.BlockSpec((1, LATENT), const),
            pl.BlockSpec((LATENT, DG), const),
            pl.BlockSpec((1, DG), const),
        ],
        out_specs=[
            pl.BlockSpec((BN, D), lambda i: (i, 0)),
            pl.BlockSpec((1, D), const),
            pl.BlockSpec((1, DG), const),
        ],
        out_shape=[
            jax.ShapeDtypeStruct((N, D), jnp.float32),
            jax.ShapeDtypeStruct((1, D), jnp.float32),
            jax.ShapeDtypeStruct((1, DG), jnp.float32),
        ],
    )(x, aggp, cntp, aggp2, cntp2, wn1x, wn1a, wn1g, bn1, wn2, bn2,
      esum, esum2, g, wg1g, wg1n, wg1e, bg1, wg2, bg2)


def kernel(x, edge_index, edge_attr, global_attr,
           We1, be1, We2, be2,
           Wn1, bn1, Wn2, bn2,
           Wg1, bg1, Wg2, bg2):
    row = edge_index[0]
    col = edge_index[1]
    g = global_attr.reshape(1, DG)

    wa = We1[:D]
    wb = We1[D:2 * D]
    wc = We1[2 * D:2 * D + DE]
    wd = We1[2 * D + DE:]

    px, qx = _project(x, wa, wb)

    e1 = (E // GRPE // 2) * GRPE
    be1r = be1.reshape(1, LATENT)
    be2r = be2.reshape(1, DE)
    halves = []
    for lo, hi in ((0, e1), (e1, E)):
        ne = hi - lo
        sh = _make_gather(ne)(px, qx, row[lo:hi], col[lo:hi])
        eoh, esh = _edge_mlp(sh, edge_attr[lo:hi], wc, wd, be1r, We2, be2r, g)
        agh, cnh = _make_scatter(ne)(eoh, col[lo:hi])
        halves.append((eoh, esh, agh, cnh))
    (eo1, es1, ag1, cn1), (eo2, es2, ag2, cn2) = halves

    wn1x = Wn1[:D]
    wn1a = Wn1[D:D + DE]
    wn1g = Wn1[D + DE:]
    wg1g = Wg1[:DG]
    wg1n = Wg1[DG:DG + D]
    wg1e = Wg1[DG + D:]

    no, _, gout = _node_global(
        x, ag1, cn1, ag2, cn2, wn1x, wn1a, wn1g, bn1.reshape(1, LATENT),
        Wn2, bn2.reshape(1, D), es1, es2, g,
        wg1g, wg1n, wg1e, bg1.reshape(1, LATENT), Wg2, bg2.reshape(1, DG))

    eo = jnp.concatenate([eo1, eo2], axis=0)
    return (no, eo, gout.reshape(DG))


# speedup vs baseline: 1.2569x; 1.2569x over previous
import functools

import jax
import jax.numpy as jnp
from jax import lax
from jax.experimental import pallas as pl
from jax.experimental.pallas import tpu as pltpu
from jax.experimental.pallas import tpu_sc as plsc

N = 10000
E = 320000
D = 128
DE = 16
DG = 16
LATENT = 128

NC = 2
NS = 16
NW = NC * NS

EPW = E // NW
CH = 80
NCHUNK = EPW // CH
NDT = 10
NPT = N // NDT

BN = 200
BE = 1280

_SC_MESH = plsc.VectorSubcoreMesh(core_axis_name="c", subcore_axis_name="s")


def _proj_body(x_ref, wa_ref, wb_ref, px_ref, qx_ref):
    x = x_ref[...]
    px_ref[...] = jnp.dot(x, wa_ref[...], preferred_element_type=jnp.float32)
    qx_ref[...] = jnp.dot(x, wb_ref[...], preferred_element_type=jnp.float32)


def _project(x, wa, wb):
    grid = (N // BN,)
    return pl.pallas_call(
        _proj_body,
        grid=grid,
        in_specs=[
            pl.BlockSpec((BN, D), lambda i: (i, 0)),
            pl.BlockSpec((D, LATENT), lambda i: (0, 0)),
            pl.BlockSpec((D, LATENT), lambda i: (0, 0)),
        ],
        out_specs=[
            pl.BlockSpec((BN, LATENT), lambda i: (i, 0)),
            pl.BlockSpec((BN, LATENT), lambda i: (i, 0)),
        ],
        out_shape=[
            jax.ShapeDtypeStruct((N, LATENT), jnp.float32),
            jax.ShapeDtypeStruct((N, LATENT), jnp.float32),
        ],
    )(x, wa, wb)


NBUF = 5
GRPE = NW * NBUF * CH


@functools.lru_cache(maxsize=None)
def _make_gather(ne):
    epw = ne // NW
    ngrp = epw // (NBUF * CH)

    @functools.partial(
        pl.kernel,
        out_type=jax.ShapeDtypeStruct((ne, LATENT), jnp.float32),
        mesh=_SC_MESH,
        scratch_types=[
            pltpu.VMEM((NBUF, CH), jnp.int32),
            pltpu.VMEM((NBUF, CH), jnp.int32),
            pltpu.VMEM((NBUF, CH, LATENT), jnp.float32),
            pltpu.SemaphoreType.DMA((NBUF,)),
            pltpu.SemaphoreType.DMA((NBUF,)),
            pltpu.SemaphoreType.DMA((NBUF,)),
        ],
    )
    def _gather_sum(px_hbm, qx_hbm, row_hbm, col_hbm, s_hbm,
                    idxa, idxb, bufs, sema, semb, semw):
        c = lax.axis_index("c")
        s = lax.axis_index("s")
        wid = s * NC + c
        base0 = wid * epw

        def group(g, carry):
            gbase = base0 + g * NBUF * CH
            for b in range(NBUF):
                base = gbase + b * CH

                @pl.when(g > 0)
                def _():
                    pltpu.make_async_copy(
                        bufs.at[b], s_hbm.at[pl.ds(base - NBUF * CH, CH)],
                        semw.at[b]).wait()

                pltpu.sync_copy(row_hbm.at[pl.ds(base, CH)], idxa.at[b])
                pltpu.sync_copy(col_hbm.at[pl.ds(base, CH)], idxb.at[b])
                pltpu.async_copy(px_hbm.at[idxa.at[b]], bufs.at[b], sema.at[b])
            for b in range(NBUF):
                pltpu.make_async_copy(px_hbm.at[idxa.at[b]], bufs.at[b],
                                      sema.at[b]).wait()
                pltpu.async_copy(qx_hbm.at[idxb.at[b]], bufs.at[b], semb.at[b],
                                 add=True)
            for b in range(NBUF):
                base = gbase + b * CH
                pltpu.make_async_copy(qx_hbm.at[idxb.at[b]], bufs.at[b],
                                      semb.at[b]).wait()
                pltpu.async_copy(bufs.at[b], s_hbm.at[pl.ds(base, CH)],
                                 semw.at[b])
            return carry

        lax.fori_loop(0, ngrp, group, 0)
        for b in range(NBUF):
            base = base0 + (ngrp - 1) * NBUF * CH + b * CH
            pltpu.make_async_copy(bufs.at[b], s_hbm.at[pl.ds(base, CH)],
                                  semw.at[b]).wait()

    return _gather_sum


def _edge_body(s_ref, eat_ref, wc_ref, wd_ref, be1_ref, w2_ref, b2_ref, g_ref,
               eo_ref, esum_ref):
    pid = pl.program_id(0)
    c0 = jnp.dot(g_ref[...], wd_ref[...], preferred_element_type=jnp.float32) + be1_ref[...]
    ea_contrib = lax.dot_general(eat_ref[...], wc_ref[...],
                                 (((0,), (0,)), ((), ())),
                                 preferred_element_type=jnp.float32)
    h = s_ref[...] + ea_contrib + c0
    h = jnp.maximum(h, 0.0)
    eo = jnp.dot(h, w2_ref[...], preferred_element_type=jnp.float32) + b2_ref[...]
    eo_ref[...] = eo

    @pl.when(pid == 0)
    def _():
        esum_ref[...] = jnp.zeros_like(esum_ref)

    esum_ref[...] += jnp.sum(eo, axis=0, keepdims=True)


def _edge_mlp(s, ea_t, wc, wd, be1, w2, b2, g):
    ne = s.shape[0]
    grid = (ne // BE,)
    return pl.pallas_call(
        _edge_body,
        grid=grid,
        in_specs=[
            pl.BlockSpec((BE, LATENT), lambda i: (i, 0)),
            pl.BlockSpec((DE, BE), lambda i: (0, i)),
            pl.BlockSpec((DE, LATENT), lambda i: (0, 0)),
            pl.BlockSpec((DG, LATENT), lambda i: (0, 0)),
            pl.BlockSpec((1, LATENT), lambda i: (0, 0)),
            pl.BlockSpec((LATENT, DE), lambda i: (0, 0)),
            pl.BlockSpec((1, DE), lambda i: (0, 0)),
            pl.BlockSpec((1, DG), lambda i: (0, 0)),
        ],
        out_specs=[
            pl.BlockSpec((BE, DE), lambda i: (i, 0)),
            pl.BlockSpec((1, DE), lambda i: (0, 0)),
        ],
        out_shape=[
            jax.ShapeDtypeStruct((ne, DE), jnp.float32),
            jax.ShapeDtypeStruct((1, DE), jnp.float32),
        ],
    )(s, ea_t, wc, wd, be1, w2, b2, g)


CHA = 40
DRN = 40


@functools.lru_cache(maxsize=None)
def _make_scatter(ne):
    epw = ne // NW
    ngrp = epw // (NBUF * CHA)

    @functools.partial(
        pl.kernel,
        out_type=jax.ShapeDtypeStruct((NC, N, DE), jnp.float32),
        mesh=_SC_MESH,
        scratch_types=[
            pltpu.VMEM((NBUF, CHA), jnp.int32),
            pltpu.VMEM((NBUF, CHA, DE), jnp.float32),
            pltpu.VMEM((DRN, DE), jnp.float32),
            pltpu.VMEM_SHARED((N, DE), jnp.float32),
            pltpu.SemaphoreType.DMA((NBUF,)),
            pltpu.SemaphoreType.DMA((NBUF,)),
            pltpu.SemaphoreType.DMA((NBUF,)),
        ],
    )
    def _segment_sum(eo_hbm, col_hbm, agg_hbm,
                     idxs, vals, stage, agg_sh, semi, semv, semadd):
        c = lax.axis_index("c")
        s = lax.axis_index("s")
        wid = s * NC + c
        base0 = wid * epw

        zeros16 = jnp.zeros((16,), jnp.float32)

        @pl.when(s < NDT)
        def _():
            def zrow(i, carry):
                stage[i, :] = zeros16
                return carry

            lax.fori_loop(0, DRN, zrow, 0, unroll=4)

            def zchunk(k, carry):
                pltpu.sync_copy(stage,
                                agg_sh.at[pl.ds(s * NPT + k * DRN, DRN)])
                return carry

            lax.fori_loop(0, NPT // DRN, zchunk, 0)

        plsc.subcore_barrier()

        def group(g, carry):
            gbase = base0 + g * NBUF * CHA
            for b in range(NBUF):
                base = gbase + b * CHA

                @pl.when(g > 0)
                def _():
                    pltpu.make_async_copy(
                        vals.at[b], agg_sh.at[idxs.at[b]], semadd.at[b]).wait()

                pltpu.async_copy(col_hbm.at[pl.ds(base, CHA)], idxs.at[b],
                                 semi.at[b])
                pltpu.async_copy(eo_hbm.at[pl.ds(base, CHA)], vals.at[b],
                                 semv.at[b])
            for b in range(NBUF):
                base = gbase + b * CHA
                pltpu.make_async_copy(col_hbm.at[pl.ds(base, CHA)],
                                      idxs.at[b], semi.at[b]).wait()
                pltpu.make_async_copy(eo_hbm.at[pl.ds(base, CHA)],
                                      vals.at[b], semv.at[b]).wait()
                pltpu.async_copy(vals.at[b], agg_sh.at[idxs.at[b]],
                                 semadd.at[b], add=True)
            return carry

        lax.fori_loop(0, ngrp, group, 0)
        for b in range(NBUF):
            pltpu.make_async_copy(vals.at[b], agg_sh.at[idxs.at[b]],
                                  semadd.at[b]).wait()
        plsc.subcore_barrier()

        @pl.when(s < NDT)
        def _():
            def dchunk(k, carry):
                off = s * NPT + k * DRN
                pltpu.sync_copy(agg_sh.at[pl.ds(off, DRN)], stage)
                pltpu.sync_copy(stage, agg_hbm.at[c, pl.ds(off, DRN)])
                return carry

            lax.fori_loop(0, NPT // DRN, dchunk, 0)

    return _segment_sum


@functools.partial(
    pl.kernel,
    out_type=jax.ShapeDtypeStruct((NC, N, DE), jnp.float32),
    mesh=_SC_MESH,
    compiler_params=pltpu.CompilerParams(use_tc_tiling_on_sc=False),
    scratch_types=[
        pltpu.VMEM((NBUF, CH), jnp.int32),
        pltpu.VMEM((CH, DE), jnp.float32),
        pltpu.VMEM((NPT, DE), jnp.float32),
        pltpu.VMEM_SHARED((N, DE), jnp.float32),
        pltpu.SemaphoreType.DMA((NBUF,)),
        pltpu.SemaphoreType.DMA((NBUF,)),
    ],
)
def _count_edges(col_hbm, cnt_hbm, idxs, onesbuf, stage, cnt_sh,
                 semi, semone):
    c = lax.axis_index("c")
    s = lax.axis_index("s")
    wid = s * NC + c
    base0 = wid * EPW

    zeros16 = jnp.zeros((16,), jnp.float32)
    ones16 = jnp.ones((16,), jnp.float32)

    def orow(i, carry):
        onesbuf[i, :] = ones16
        return carry

    lax.fori_loop(0, CH, orow, 0, unroll=4)

    @pl.when(s < NDT)
    def _():
        def zrow(i, carry):
            stage[i, :] = zeros16
            return carry

        lax.fori_loop(0, NPT, zrow, 0, unroll=4)
        pltpu.sync_copy(stage, cnt_sh.at[pl.ds(s * NPT, NPT)])

    plsc.subcore_barrier()

    def group(g, carry):
        gbase = base0 + g * NBUF * CH
        for b in range(NBUF):
            base = gbase + b * CH

            @pl.when(g > 0)
            def _():
                pltpu.make_async_copy(
                    onesbuf, cnt_sh.at[idxs.at[b]], semone.at[b]).wait()

            pltpu.async_copy(col_hbm.at[pl.ds(base, CH)], idxs.at[b],
                             semi.at[b])
        for b in range(NBUF):
            base = gbase + b * CH
            pltpu.make_async_copy(col_hbm.at[pl.ds(base, CH)], idxs.at[b],
                                  semi.at[b]).wait()
            pltpu.async_copy(onesbuf, cnt_sh.at[idxs.at[b]], semone.at[b],
                             add=True)
        return carry

    lax.fori_loop(0, NCHUNK // NBUF, group, 0)
    for b in range(NBUF):
        pltpu.make_async_copy(onesbuf, cnt_sh.at[idxs.at[b]],
                              semone.at[b]).wait()
    plsc.subcore_barrier()

    @pl.when(s < NDT)
    def _():
        pltpu.sync_copy(cnt_sh.at[pl.ds(s * NPT, NPT)], stage)
        pltpu.sync_copy(stage, cnt_hbm.at[c, pl.ds(s * NPT, NPT)])


def _node_body(x_ref, aggp_ref, aggp2_ref, cntp_ref,
               wn1x_ref, wn1a_ref, wn1g_ref,
               bn1_ref, wn2_ref, bn2_ref, esum_ref, esum2_ref, g_ref,
               wg1g_ref, wg1n_ref, wg1e_ref, bg1_ref, wg2_ref, bg2_ref,
               no_ref, nsum_ref, gout_ref):
    pid = pl.program_id(0)
    nblocks = pl.num_programs(0)
    ap = aggp_ref[0] + aggp_ref[1] + aggp2_ref[0] + aggp2_ref[1]
    cp = cntp_ref[0] + cntp_ref[1]
    agg = ap / jnp.maximum(cp, 1.0)
    cn0 = jnp.dot(g_ref[...], wn1g_ref[...], preferred_element_type=jnp.float32) + bn1_ref[...]
    h = (jnp.dot(x_ref[...], wn1x_ref[...], preferred_element_type=jnp.float32)
         + jnp.dot(agg, wn1a_ref[...], preferred_element_type=jnp.float32) + cn0)
    h = jnp.maximum(h, 0.0)
    no = jnp.dot(h, wn2_ref[...], preferred_element_type=jnp.float32) + bn2_ref[...]
    no_ref[...] = no

    @pl.when(pid == 0)
    def _():
        nsum_ref[...] = jnp.zeros_like(nsum_ref)

    nsum_ref[...] += jnp.sum(no, axis=0, keepdims=True)

    @pl.when(pid == nblocks - 1)
    def _():
        node_mean = nsum_ref[...] * (1.0 / N)
        edge_mean = (esum_ref[...] + esum2_ref[...]) * (1.0 / E)
        hg = (jnp.dot(g_ref[...], wg1g_ref[...], preferred_element_type=jnp.float32)
              + jnp.dot(node_mean, wg1n_ref[...], preferred_element_type=jnp.float32)
              + jnp.dot(edge_mean, wg1e_ref[...], preferred_element_type=jnp.float32)
              + bg1_ref[...])
        hg = jnp.maximum(hg, 0.0)
        gout_ref[...] = jnp.dot(hg, wg2_ref[...], preferred_element_type=jnp.float32) + bg2_ref[...]


def _node_global(x, aggp, aggp2, cntp, wn1x, wn1a, wn1g, bn1, wn2, bn2,
                 esum, esum2, g, wg1g, wg1n, wg1e, bg1, wg2, bg2):
    grid = (N // BN,)
    const = lambda i: (0, 0)
    return pl.pallas_call(
        _node_body,
        grid=grid,
        in_specs=[
            pl.BlockSpec((BN, D), lambda i: (i, 0)),
            pl.BlockSpec((NC, BN, DE), lambda i: (0, i, 0)),
            pl.BlockSpec((NC, BN, DE), lambda i: (0, i, 0)),
            pl.BlockSpec((NC, BN, DE), lambda i: (0, i, 0)),
            pl.BlockSpec((D, LATENT), const),
            pl.BlockSpec((DE, LATENT), const),
            pl.BlockSpec((DG, LATENT), const),
            pl.BlockSpec((1, LATENT), const),
            pl.BlockSpec((LATENT, D), const),
            pl.BlockSpec((1, D), const),
            pl.BlockSpec((1, DE), const),
            pl.BlockSpec((1, DE), const),
            pl.BlockSpec((1, DG), const),
            pl.BlockSpec((DG, LATENT), const),
            pl.BlockSpec((D, LATENT), const),
            pl.BlockSpec((DE, LATENT), const),
            pl.BlockSpec((1, LATENT), const),
            pl.BlockSpec((LATENT, DG), const),
            pl.BlockSpec((1, DG), const),
        ],
        out_specs=[
            pl.BlockSpec((BN, D), lambda i: (i, 0)),
            pl.BlockSpec((1, D), const),
            pl.BlockSpec((1, DG), const),
        ],
        out_shape=[
            jax.ShapeDtypeStruct((N, D), jnp.float32),
            jax.ShapeDtypeStruct((1, D), jnp.float32),
            jax.ShapeDtypeStruct((1, DG), jnp.float32),
        ],
    )(x, aggp, aggp2, cntp, wn1x, wn1a, wn1g, bn1, wn2, bn2,
      esum, esum2, g, wg1g, wg1n, wg1e, bg1, wg2, bg2)


def kernel(x, edge_index, edge_attr, global_attr,
           We1, be1, We2, be2,
           Wn1, bn1, Wn2, bn2,
           Wg1, bg1, Wg2, bg2):
    row = edge_index[0]
    col = edge_index[1]
    g = global_attr.reshape(1, DG)

    wa = We1[:D]
    wb = We1[D:2 * D]
    wc = We1[2 * D:2 * D + DE]
    wd = We1[2 * D + DE:]

    px, qx = _project(x, wa, wb)

    e1 = (E // GRPE // 2) * GRPE
    ea_t = edge_attr.T
    be1r = be1.reshape(1, LATENT)
    be2r = be2.reshape(1, DE)
    cnt = _count_edges(col)
    halves = []
    for lo, hi in ((0, e1), (e1, E)):
        ne = hi - lo
        sh = _make_gather(ne)(px, qx, row[lo:hi], col[lo:hi])
        eoh, esh = _edge_mlp(sh, ea_t[:, lo:hi], wc, wd, be1r, We2, be2r, g)
        agh = _make_scatter(ne)(eoh, col[lo:hi])
        halves.append((eoh, esh, agh))
    (eo1, es1, ag1), (eo2, es2, ag2) = halves

    wn1x = Wn1[:D]
    wn1a = Wn1[D:D + DE]
    wn1g = Wn1[D + DE:]
    wg1g = Wg1[:DG]
    wg1n = Wg1[DG:DG + D]
    wg1e = Wg1[DG + D:]

    no, _, gout = _node_global(
        x, ag1, ag2, cnt, wn1x, wn1a, wn1g, bn1.reshape(1, LATENT),
        Wn2, bn2.reshape(1, D), es1, es2, g,
        wg1g, wg1n, wg1e, bg1.reshape(1, LATENT), Wg2, bg2.reshape(1, DG))

    eo = jnp.concatenate([eo1, eo2], axis=0)
    return (no, eo, gout.reshape(DG))
